# baseline (speedup 1.0000x reference)
import functools

import jax
import jax.numpy as jnp
from jax import lax
from jax.experimental import pallas as pl
from jax.experimental.pallas import tpu as pltpu
from jax.experimental.pallas import tpu_sc as plsc

_NC = 2
_NS = 16
_LANES = 16
_CHUNK = 4


def _make_sc_kernel(S, B, D, dtype):
    NW = _NC * _NS
    rows_per_w = S // NW
    n_chunks = rows_per_w // _CHUNK
    mesh = plsc.VectorSubcoreMesh(core_axis_name="c", subcore_axis_name="s")

    @functools.partial(
        pl.kernel,
        out_type=jax.ShapeDtypeStruct((S, B, D), dtype),
        mesh=mesh,
        scratch_types=[
            pltpu.VMEM((_CHUNK, B, D), dtype),
            pltpu.VMEM((_CHUNK, B, D), dtype),
            pltpu.VMEM((_CHUNK, D), dtype),
            pltpu.VMEM((_CHUNK, D), dtype),
            pltpu.VMEM((_CHUNK, B, D), dtype),
            pltpu.VMEM((_CHUNK, B, D), dtype),
            pltpu.SemaphoreType.DMA,
            pltpu.SemaphoreType.DMA,
            pltpu.SemaphoreType.DMA,
            pltpu.SemaphoreType.DMA,
            pltpu.SemaphoreType.DMA,
            pltpu.SemaphoreType.DMA,
        ],
    )
    def k(x_hbm, pe_hbm, out_hbm, xb0, xb1, pb0, pb1, ob0, ob1,
          si0, si1, sp0, sp1, so0, so1):
        wid = lax.axis_index("s") * _NC + lax.axis_index("c")
        base = wid * rows_per_w
        xbufs, pbufs, obufs = [xb0, xb1], [pb0, pb1], [ob0, ob1]
        sin, spe, sout = [si0, si1], [sp0, sp1], [so0, so1]
        in_d = [None] * n_chunks
        pe_d = [None] * n_chunks
        out_d = [None] * n_chunks

        def start_in(ci):
            row = base + ci * _CHUNK
            b = ci % 2
            in_d[ci] = pltpu.async_copy(
                x_hbm.at[pl.ds(row, _CHUNK)], xbufs[b], sin[b])
            pe_d[ci] = pltpu.async_copy(
                pe_hbm.at[pl.ds(row, _CHUNK)], pbufs[b], spe[b])

        start_in(0)
        for ci in range(n_chunks):
            b = ci % 2
            if ci + 1 < n_chunks:
                start_in(ci + 1)
            in_d[ci].wait()
            pe_d[ci].wait()
            if ci >= 2:
                out_d[ci - 2].wait()
            xbuf, pbuf, obuf = xbufs[b], pbufs[b], obufs[b]
            for r in range(_CHUNK):
                @plsc.parallel_loop(0, D, _LANES, unroll=4)
                def d_body(dd, r=r, xbuf=xbuf, pbuf=pbuf, obuf=obuf):
                    sl = pl.ds(dd, _LANES)
                    pv = pbuf[r, sl]
                    for bb in range(B):
                        obuf[r, bb, sl] = xbuf[r, bb, sl] + pv
            row = base + ci * _CHUNK
            out_d[ci] = pltpu.async_copy(
                obuf, out_hbm.at[pl.ds(row, _CHUNK)], sout[b])
        out_d[n_chunks - 2].wait()
        out_d[n_chunks - 1].wait()

    return k


def kernel(x, pe):
    S, B, D = x.shape
    return _make_sc_kernel(S, B, D, x.dtype)(x, pe[:S])


# speedup vs baseline: 2.2525x; 2.2525x over previous
import jax
import jax.numpy as jnp
from jax.experimental import pallas as pl


_BS = 512


def _add_body(x_ref, pe_ref, o_ref):
    o_ref[...] = x_ref[...] + pe_ref[...][:, None, :]


def kernel(x, pe):
    S, B, D = x.shape
    grid = (S // _BS,)
    return pl.pallas_call(
        _add_body,
        grid=grid,
        in_specs=[
            pl.BlockSpec((_BS, B, D), lambda i: (i, 0, 0)),
            pl.BlockSpec((_BS, D), lambda i: (i, 0)),
        ],
        out_specs=pl.BlockSpec((_BS, B, D), lambda i: (i, 0, 0)),
        out_shape=jax.ShapeDtypeStruct((S, B, D), x.dtype),
    )(x, pe[:S])
